# Initial kernel scaffold; baseline (speedup 1.0000x reference)
import jax
import jax.numpy as jnp
from jax.experimental import pallas as pl


def kernel(x, W):
    raise NotImplementedError("write your pallas kernel here")


# speedup vs baseline: 1.7334x; 1.7334x over previous
import functools

import jax
import jax.numpy as jnp
from jax.experimental import pallas as pl

N_EXPERTS = 64
N_ACT = 8
BLOCK_ROWS = 512


def _gate_kernel(x_ref, wt_ref, wout_ref, iout_ref):
    x = x_ref[...]
    wt = wt_ref[...]
    scores = jnp.dot(x, wt, preferred_element_type=jnp.float32)
    m = jnp.max(scores, axis=-1, keepdims=True)
    e = jnp.exp(scores - m)
    p = e / jnp.sum(e, axis=-1, keepdims=True)

    rows = p.shape[0]
    col = jax.lax.broadcasted_iota(jnp.int32, (rows, N_EXPERTS), 1)
    vals = []
    idxs = []
    cur = p
    for _ in range(N_ACT):
        v = jnp.max(cur, axis=-1, keepdims=True)
        i = jnp.argmax(cur, axis=-1)
        vals.append(v)
        idxs.append(i[:, None])
        cur = jnp.where(col == i[:, None], -jnp.inf, cur)
    wout_ref[...] = jnp.concatenate(vals, axis=-1)
    iout_ref[...] = jnp.concatenate(idxs, axis=-1).astype(jnp.int32)


@jax.jit
def kernel(x, W):
    n_rows = x.shape[0]
    wt = W.T
    grid = (n_rows // BLOCK_ROWS,)
    weights, indices = pl.pallas_call(
        _gate_kernel,
        grid=grid,
        in_specs=[
            pl.BlockSpec((BLOCK_ROWS, x.shape[1]), lambda i: (i, 0)),
            pl.BlockSpec((x.shape[1], N_EXPERTS), lambda i: (0, 0)),
        ],
        out_specs=[
            pl.BlockSpec((BLOCK_ROWS, N_ACT), lambda i: (i, 0)),
            pl.BlockSpec((BLOCK_ROWS, N_ACT), lambda i: (i, 0)),
        ],
        out_shape=[
            jax.ShapeDtypeStruct((n_rows, N_ACT), jnp.float32),
            jax.ShapeDtypeStruct((n_rows, N_ACT), jnp.int32),
        ],
    )(x, wt)
    return weights, indices
